# baseline (speedup 1.0000x reference)
import functools

import jax
import jax.numpy as jnp
from jax.experimental import pallas as pl
from jax.experimental.pallas import tpu as pltpu


def _mlp2_body(x_ref, w1_ref, b1_ref, w2_ref, b2_ref, a_ref, o_ref):
    h = jnp.dot(x_ref[...], w1_ref[...], preferred_element_type=jnp.float32)
    h = h + b1_ref[...]
    a0 = a_ref[0, 0]
    h = jnp.where(h >= 0, h, a0 * h)
    y = jnp.dot(h, w2_ref[...], preferred_element_type=jnp.float32)
    y = y + b2_ref[...]
    a1 = a_ref[0, 1]
    o_ref[...] = jnp.where(y >= 0, y, a1 * y)


def _fused_mlp(x, w1, b1, w2, b2, a01, block_rows):
    rows, d = x.shape
    grid = (rows // block_rows,)
    return pl.pallas_call(
        _mlp2_body,
        grid=grid,
        in_specs=[
            pl.BlockSpec((block_rows, d), lambda i: (i, 0)),
            pl.BlockSpec((d, d), lambda i: (0, 0)),
            pl.BlockSpec((1, d), lambda i: (0, 0)),
            pl.BlockSpec((d, d), lambda i: (0, 0)),
            pl.BlockSpec((1, d), lambda i: (0, 0)),
            pl.BlockSpec(memory_space=pltpu.SMEM),
        ],
        out_specs=pl.BlockSpec((block_rows, d), lambda i: (i, 0)),
        out_shape=jax.ShapeDtypeStruct((rows, d), jnp.float32),
    )(x, w1, b1, w2, b2, a01)


@jax.jit
def kernel(node_feats, edge_feats, adj_matrix, pn_W, pn_b, pe_W, pe_b,
           em_W0, em_b0, em_a0, em_W1, em_b1, em_a1,
           nm_W0, nm_b0, nm_a0, nm_W1, nm_b1, nm_a1):
    num_edges = edge_feats.shape[0]

    wn1 = (nm_W0 @ pn_W).T
    bn1 = (nm_W0 @ pn_b + nm_b0).reshape(1, -1)
    wn2 = nm_W1.T
    bn2 = nm_b1.reshape(1, -1)
    an = jnp.stack([nm_a0, nm_a1]).reshape(1, 2)

    eye8 = jnp.eye(8, dtype=jnp.float32)
    we1 = jnp.kron(eye8, (em_W0 @ pe_W).T)
    be1 = jnp.tile(em_W0 @ pe_b + em_b0, 8).reshape(1, -1)
    we2 = jnp.kron(eye8, em_W1.T)
    be2 = jnp.tile(em_b1, 8).reshape(1, -1)
    ae = jnp.stack([em_a0, em_a1]).reshape(1, 2)

    n = _fused_mlp(node_feats, wn1, bn1, wn2, bn2, an, block_rows=2000)

    x2 = edge_feats.reshape(num_edges // 8, 128)
    e2 = _fused_mlp(x2, we1, be1, we2, be2, ae, block_rows=4000)
    e = e2.reshape(num_edges, 16)

    return (n, e)


# speedup vs baseline: 6.6115x; 6.6115x over previous
import jax
import jax.numpy as jnp
from jax.experimental import pallas as pl
from jax.experimental.pallas import tpu as pltpu


def _node_body(x_ref, w1_ref, b1_ref, w2_ref, b2_ref, a_ref, o_ref):
    h = jnp.dot(x_ref[...], w1_ref[...], preferred_element_type=jnp.float32)
    h = h + b1_ref[...]
    a0 = a_ref[0, 0]
    h = jnp.where(h >= 0, h, a0 * h)
    y = jnp.dot(h, w2_ref[...], preferred_element_type=jnp.float32)
    y = y + b2_ref[...]
    a1 = a_ref[0, 1]
    o_ref[...] = jnp.where(y >= 0, y, a1 * y)


def _edge_body(x_ref, w1_ref, b1_ref, w2_ref, b2_ref, a_ref, o_ref):
    h = jnp.dot(w1_ref[...], x_ref[...], preferred_element_type=jnp.float32)
    h = h + b1_ref[...]
    a0 = a_ref[0, 0]
    h = jnp.where(h >= 0, h, a0 * h)
    y = jnp.dot(w2_ref[...], h, preferred_element_type=jnp.float32)
    y = y + b2_ref[...]
    a1 = a_ref[0, 1]
    o_ref[...] = jnp.where(y >= 0, y, a1 * y)


@jax.jit
def kernel(node_feats, edge_feats, adj_matrix, pn_W, pn_b, pe_W, pe_b,
           em_W0, em_b0, em_a0, em_W1, em_b1, em_a1,
           nm_W0, nm_b0, nm_a0, nm_W1, nm_b1, nm_a1):
    num_nodes = node_feats.shape[0]
    num_edges = edge_feats.shape[0]

    wn1 = (nm_W0 @ pn_W).T
    bn1 = (nm_W0 @ pn_b + nm_b0).reshape(1, -1)
    wn2 = nm_W1.T
    bn2 = nm_b1.reshape(1, -1)
    an = jnp.stack([nm_a0, nm_a1]).reshape(1, 2)

    we1 = em_W0 @ pe_W
    be1 = (em_W0 @ pe_b + em_b0).reshape(-1, 1)
    we2 = em_W1
    be2 = em_b1.reshape(-1, 1)
    ae = jnp.stack([em_a0, em_a1]).reshape(1, 2)

    bn = 2000
    n = pl.pallas_call(
        _node_body,
        grid=(num_nodes // bn,),
        in_specs=[
            pl.BlockSpec((bn, 128), lambda i: (i, 0)),
            pl.BlockSpec((128, 128), lambda i: (0, 0)),
            pl.BlockSpec((1, 128), lambda i: (0, 0)),
            pl.BlockSpec((128, 128), lambda i: (0, 0)),
            pl.BlockSpec((1, 128), lambda i: (0, 0)),
            pl.BlockSpec(memory_space=pltpu.SMEM),
        ],
        out_specs=pl.BlockSpec((bn, 128), lambda i: (i, 0)),
        out_shape=jax.ShapeDtypeStruct((num_nodes, 128), jnp.float32),
    )(node_feats, wn1, bn1, wn2, bn2, an)

    xt = edge_feats.T
    be = 16000
    et = pl.pallas_call(
        _edge_body,
        grid=(num_edges // be,),
        in_specs=[
            pl.BlockSpec((16, be), lambda i: (0, i)),
            pl.BlockSpec((16, 16), lambda i: (0, 0)),
            pl.BlockSpec((16, 1), lambda i: (0, 0)),
            pl.BlockSpec((16, 16), lambda i: (0, 0)),
            pl.BlockSpec((16, 1), lambda i: (0, 0)),
            pl.BlockSpec(memory_space=pltpu.SMEM),
        ],
        out_specs=pl.BlockSpec((16, be), lambda i: (0, i)),
        out_shape=jax.ShapeDtypeStruct((16, num_edges), jnp.float32),
    )(xt, we1, be1, we2, be2, ae)
    e = et.T

    return (n, e)


# speedup vs baseline: 8.7145x; 1.3181x over previous
import jax
import jax.numpy as jnp
from jax import lax
from jax.experimental import pallas as pl
from jax.experimental.pallas import tpu as pltpu

_NBN = 5
_BN = 2000
_NBE = 20
_BE = 16000


def _dot_t(a, b):
    return lax.dot_general(a, b, (((1,), (1,)), ((), ())),
                           preferred_element_type=jnp.float32)


def _body(x_ref, xt_ref, pn_W_ref, pn_b_ref, pe_W_ref, pe_b_ref,
          em_W0_ref, em_b0_ref, em_W1_ref, em_b1_ref,
          nm_W0_ref, nm_b0_ref, nm_W1_ref, nm_b1_ref,
          ea0_ref, ea1_ref, na0_ref, na1_ref,
          n_ref, et_ref):
    i = pl.program_id(0)

    @pl.when(i < _NBN)
    def _node():
        wf = jnp.dot(nm_W0_ref[...], pn_W_ref[...],
                     preferred_element_type=jnp.float32)
        b1 = _dot_t(pn_b_ref[...], nm_W0_ref[...]) + nm_b0_ref[...]
        h = _dot_t(x_ref[...], wf) + b1
        a0 = na0_ref[0, 0]
        h = jnp.where(h >= 0, h, a0 * h)
        y = _dot_t(h, nm_W1_ref[...]) + nm_b1_ref[...]
        a1 = na1_ref[0, 0]
        n_ref[...] = jnp.where(y >= 0, y, a1 * y)

    @pl.when(i >= _NBN)
    def _edge():
        eye = jnp.eye(16, dtype=jnp.float32)
        wf = jnp.dot(em_W0_ref[...], pe_W_ref[...],
                     preferred_element_type=jnp.float32)
        b1 = _dot_t(em_W0_ref[...], pe_b_ref[...]) + _dot_t(eye, em_b0_ref[...])
        b2 = _dot_t(eye, em_b1_ref[...])
        h = jnp.dot(wf, xt_ref[...], preferred_element_type=jnp.float32) + b1
        a0 = ea0_ref[0, 0]
        h = jnp.where(h >= 0, h, a0 * h)
        y = jnp.dot(em_W1_ref[...], h, preferred_element_type=jnp.float32) + b2
        a1 = ea1_ref[0, 0]
        et_ref[...] = jnp.where(y >= 0, y, a1 * y)


@jax.jit
def kernel(node_feats, edge_feats, adj_matrix, pn_W, pn_b, pe_W, pe_b,
           em_W0, em_b0, em_a0, em_W1, em_b1, em_a1,
           nm_W0, nm_b0, nm_a0, nm_W1, nm_b1, nm_a1):
    num_nodes = node_feats.shape[0]
    num_edges = edge_feats.shape[0]

    xt = edge_feats.T

    full = lambda shape: pl.BlockSpec(shape, lambda i: (0, 0))
    smem = pl.BlockSpec(memory_space=pltpu.SMEM)

    n, et = pl.pallas_call(
        _body,
        grid=(_NBN + _NBE,),
        in_specs=[
            pl.BlockSpec((_BN, 128), lambda i: (jnp.minimum(i, _NBN - 1), 0)),
            pl.BlockSpec((16, _BE), lambda i: (0, jnp.maximum(i - _NBN, 0))),
            full((128, 128)),
            full((1, 128)),
            full((16, 16)),
            full((1, 16)),
            full((16, 16)),
            full((1, 16)),
            full((16, 16)),
            full((1, 16)),
            full((128, 128)),
            full((1, 128)),
            full((128, 128)),
            full((1, 128)),
            smem, smem, smem, smem,
        ],
        out_specs=[
            pl.BlockSpec((_BN, 128), lambda i: (jnp.minimum(i, _NBN - 1), 0)),
            pl.BlockSpec((16, _BE), lambda i: (0, jnp.maximum(i - _NBN, 0))),
        ],
        out_shape=[
            jax.ShapeDtypeStruct((num_nodes, 128), jnp.float32),
            jax.ShapeDtypeStruct((16, num_edges), jnp.float32),
        ],
    )(node_feats, xt, pn_W, pn_b.reshape(1, -1), pe_W, pe_b.reshape(1, -1),
      em_W0, em_b0.reshape(1, -1), em_W1, em_b1.reshape(1, -1),
      nm_W0, nm_b0.reshape(1, -1), nm_W1, nm_b1.reshape(1, -1),
      em_a0.reshape(1, 1), em_a1.reshape(1, 1),
      nm_a0.reshape(1, 1), nm_a1.reshape(1, 1))

    return (n, et.T)
